# baseline (speedup 1.0000x reference)
import jax
import jax.numpy as jnp
from jax.experimental import pallas as pl

_N_GT = 100
_N_CLASSES = 21
_COLS = 4 * _N_CLASSES


def _ptl_body(rois_ref, gt5_ref, lab_ref, tgt_ref, inw_ref):
    roist = jnp.transpose(rois_ref[...])
    gt_t = jnp.transpose(gt5_ref[...])
    x1 = roist[1:2, :]
    y1 = roist[2:3, :]
    x2 = roist[3:4, :]
    y2 = roist[4:5, :]
    gx1 = gt5_ref[:, 0:1]
    gy1 = gt5_ref[:, 1:2]
    gx2 = gt5_ref[:, 2:3]
    gy2 = gt5_ref[:, 3:4]

    area_b = (x2 - x1 + 1.0) * (y2 - y1 + 1.0)
    area_g = (gx2 - gx1 + 1.0) * (gy2 - gy1 + 1.0)
    iw = jnp.clip(jnp.minimum(x2, gx2) - jnp.maximum(x1, gx1) + 1.0, 0.0)
    ih = jnp.clip(jnp.minimum(y2, gy2) - jnp.maximum(y1, gy1) + 1.0, 0.0)
    inter = iw * ih
    ov = inter / (area_b + area_g - inter)

    max_ov = jnp.max(ov, axis=0, keepdims=True)
    gt_iota = jax.lax.broadcasted_iota(jnp.int32, (_N_GT, 1), 0)
    idx = jnp.min(jnp.where(ov == max_ov, gt_iota, _N_GT), axis=0, keepdims=True)
    fg = max_ov >= 0.5
    onehot = (gt_iota == idx).astype(jnp.float32)

    assigned = jax.lax.dot_general(
        gt_t, onehot, (((1,), (0,)), ((), ())),
        precision=jax.lax.Precision.HIGHEST,
        preferred_element_type=jnp.float32)
    ax1 = assigned[0:1, :]
    ay1 = assigned[1:2, :]
    ax2 = assigned[2:3, :]
    ay2 = assigned[3:4, :]
    alab = assigned[4:5, :]

    lab_row = jnp.where(fg, alab, 0.0)
    lab_ref[...] = lab_row.reshape(lab_ref.shape)

    ew = x2 - x1 + 1.0
    eh = y2 - y1 + 1.0
    ecx = x1 + 0.5 * ew
    ecy = y1 + 0.5 * eh
    gw = ax2 - ax1 + 1.0
    gh = ay2 - ay1 + 1.0
    gcx = ax1 + 0.5 * gw
    gcy = ay1 + 0.5 * gh
    dx = ((gcx - ecx) / ew) / 0.1
    dy = ((gcy - ecy) / eh) / 0.1
    dw = jnp.log(gw / ew) / 0.2
    dh = jnp.log(gh / eh) / 0.2
    fgf = jnp.where(fg, 1.0, 0.0)

    pack = jnp.concatenate(
        [dx, dy, dw, dh, lab_row, fgf, fgf, fgf], axis=0)
    packc = jnp.transpose(pack)
    dxc = packc[:, 0:1]
    dyc = packc[:, 1:2]
    dwc = packc[:, 2:3]
    dhc = packc[:, 3:4]
    cls = packc[:, 4:5].astype(jnp.int32)
    fgc = packc[:, 5:6] > 0.5

    colcls = jax.lax.broadcasted_iota(jnp.int32, (1, _COLS), 1) // 4
    jmod = jax.lax.broadcasted_iota(jnp.int32, (1, _COLS), 1) % 4
    m = (colcls == cls) & fgc
    t = jnp.where(jmod == 0, dxc,
                  jnp.where(jmod == 1, dyc,
                            jnp.where(jmod == 2, dwc, dhc)))
    tgt_ref[...] = jnp.where(m, t, 0.0)
    inw_ref[...] = jnp.where(m, 1.0, 0.0)


def kernel(all_rois, gt_boxes, block_rows: int = 2000, interpret: bool = False):
    n = all_rois.shape[0]
    g = n // block_rows
    grid = (g,)
    labels, tgt, inw = pl.pallas_call(
        _ptl_body,
        grid=grid,
        in_specs=[
            pl.BlockSpec((block_rows, 5), lambda i: (i, 0)),
            pl.BlockSpec((_N_GT, 5), lambda i: (0, 0)),
        ],
        out_specs=[
            pl.BlockSpec((1, 1, block_rows), lambda i: (i, 0, 0)),
            pl.BlockSpec((block_rows, _COLS), lambda i: (i, 0)),
            pl.BlockSpec((block_rows, _COLS), lambda i: (i, 0)),
        ],
        out_shape=[
            jax.ShapeDtypeStruct((g, 1, block_rows), jnp.float32),
            jax.ShapeDtypeStruct((n, _COLS), jnp.float32),
            jax.ShapeDtypeStruct((n, _COLS), jnp.float32),
        ],
        interpret=interpret,
    )(all_rois, gt_boxes)
    return labels.reshape((n,)), tgt, inw


# speedup vs baseline: 1.3051x; 1.3051x over previous
import jax
import jax.numpy as jnp
from jax.experimental import pallas as pl

_N_GT = 100
_N_CLASSES = 21
_COLS = 4 * _N_CLASSES


def _ptl_body(roist_ref, gt_ref, gt5_ref, lab_ref, tgt_ref, inw_ref):
    x1 = roist_ref[0, 1:2, :]
    y1 = roist_ref[0, 2:3, :]
    x2 = roist_ref[0, 3:4, :]
    y2 = roist_ref[0, 4:5, :]
    gx1 = gt5_ref[:, 0:1]
    gy1 = gt5_ref[:, 1:2]
    gx2 = gt5_ref[:, 2:3]
    gy2 = gt5_ref[:, 3:4]

    area_b = (x2 - x1 + 1.0) * (y2 - y1 + 1.0)
    area_g = (gx2 - gx1 + 1.0) * (gy2 - gy1 + 1.0)
    iw = jnp.clip(jnp.minimum(x2, gx2) - jnp.maximum(x1, gx1) + 1.0, 0.0)
    ih = jnp.clip(jnp.minimum(y2, gy2) - jnp.maximum(y1, gy1) + 1.0, 0.0)
    inter = iw * ih
    ov = inter / (area_b + area_g - inter)

    max_ov = jnp.max(ov, axis=0, keepdims=True)
    gt_iota = jax.lax.broadcasted_iota(jnp.int32, (_N_GT, 1), 0)
    idx = jnp.min(jnp.where(ov == max_ov, gt_iota, _N_GT), axis=0, keepdims=True)
    fg = max_ov >= 0.5
    onehot = (gt_iota == idx).astype(jnp.float32)

    assigned = jax.lax.dot_general(
        gt_ref[...], onehot, (((1,), (0,)), ((), ())),
        precision=jax.lax.Precision.HIGHEST,
        preferred_element_type=jnp.float32)
    ax1 = assigned[0:1, :]
    ay1 = assigned[1:2, :]
    ax2 = assigned[2:3, :]
    ay2 = assigned[3:4, :]
    alab = assigned[4:5, :]

    lab_row = jnp.where(fg, alab, 0.0)
    lab_ref[...] = lab_row.reshape(lab_ref.shape)

    ew = x2 - x1 + 1.0
    eh = y2 - y1 + 1.0
    ecx = x1 + 0.5 * ew
    ecy = y1 + 0.5 * eh
    gw = ax2 - ax1 + 1.0
    gh = ay2 - ay1 + 1.0
    gcx = ax1 + 0.5 * gw
    gcy = ay1 + 0.5 * gh
    dx = ((gcx - ecx) / ew) / 0.1
    dy = ((gcy - ecy) / eh) / 0.1
    dw = jnp.log(gw / ew) / 0.2
    dh = jnp.log(gh / eh) / 0.2
    clsm = jnp.where(fg, alab, -1.0)

    pack = jnp.concatenate(
        [dx, dy, dw, dh, clsm, clsm, clsm, clsm], axis=0)
    packc = jnp.transpose(pack)
    t4 = packc[:, 0:4]
    cls = packc[:, 4:5].astype(jnp.int32)

    colcls = jax.lax.broadcasted_iota(jnp.int32, (1, _COLS), 1) // 4
    jmod = jax.lax.broadcasted_iota(jnp.int32, (1, _COLS), 1) % 4
    m_f = (colcls == cls).astype(jnp.float32)
    q_iota = jax.lax.broadcasted_iota(jnp.int32, (4, 1), 0)
    q = (q_iota == jmod).astype(jnp.float32)
    t_rep = jax.lax.dot_general(
        t4, q, (((1,), (0,)), ((), ())),
        precision=jax.lax.Precision.HIGHEST,
        preferred_element_type=jnp.float32)
    tgt_ref[...] = m_f * t_rep
    inw_ref[...] = m_f


def kernel(all_rois, gt_boxes, block_rows: int = 2000, interpret: bool = False):
    n = all_rois.shape[0]
    g = n // block_rows
    rois_t = all_rois.reshape(g, block_rows, 5).transpose(0, 2, 1)
    gt_t = gt_boxes.T
    grid = (g,)
    labels, tgt, inw = pl.pallas_call(
        _ptl_body,
        grid=grid,
        in_specs=[
            pl.BlockSpec((1, 5, block_rows), lambda i: (i, 0, 0)),
            pl.BlockSpec((5, _N_GT), lambda i: (0, 0)),
            pl.BlockSpec((_N_GT, 5), lambda i: (0, 0)),
        ],
        out_specs=[
            pl.BlockSpec((1, 1, block_rows), lambda i: (i, 0, 0)),
            pl.BlockSpec((block_rows, _COLS), lambda i: (i, 0)),
            pl.BlockSpec((block_rows, _COLS), lambda i: (i, 0)),
        ],
        out_shape=[
            jax.ShapeDtypeStruct((g, 1, block_rows), jnp.float32),
            jax.ShapeDtypeStruct((n, _COLS), jnp.float32),
            jax.ShapeDtypeStruct((n, _COLS), jnp.float32),
        ],
        interpret=interpret,
    )(rois_t, gt_t, gt_boxes)
    return labels.reshape((n,)), tgt, inw
